# baseline (speedup 1.0000x reference)
import functools

import jax
import jax.numpy as jnp
from jax import lax
from jax.experimental import pallas as pl
from jax.experimental.pallas import tpu as pltpu
from jax.experimental.pallas import tpu_sc as plsc

NUM_USERS = 1000000
BATCH = 16384
EMB_DIM = 64
FEAT_DIM = 32
HID1 = 256
HID2 = 128
OUT_DIM = 64

_SC_INFO = plsc.get_sparse_core_info()
_NUM_WORKERS = _SC_INFO.num_cores * _SC_INFO.num_subcores
_B_PER_W = BATCH // _NUM_WORKERS
_WINDOW = 64

_PACK_TB = 16384


_N_PACK_BLOCKS = -(-NUM_USERS // (2 * _PACK_TB))
_PACKED_ROWS = _N_PACK_BLOCKS * _PACK_TB


def _pack_body(lo_ref, hi_ref, eye_ref, out_ref):
    cdim0 = (((0,), (0,)), ((), ()))
    eye = eye_ref[...]
    i = pl.program_id(0)
    valid = NUM_USERS - (2 * i + 1) * _PACK_TB
    cols = lax.broadcasted_iota(jnp.int32, (EMB_DIM, _PACK_TB), 1)
    hi = jnp.where(cols < valid, hi_ref[...], 0.0)
    out_ref[...] = (
        lax.dot_general(lo_ref[...], eye[:, :128], cdim0,
                        preferred_element_type=jnp.float32)
        + lax.dot_general(hi, eye[:, 128:], cdim0,
                          preferred_element_type=jnp.float32))


def _pack_table_tc(tableT, eye):
    return pl.pallas_call(
        _pack_body,
        grid=(_N_PACK_BLOCKS,),
        in_specs=[
            pl.BlockSpec((EMB_DIM, _PACK_TB), lambda i: (0, 2 * i)),
            pl.BlockSpec((EMB_DIM, _PACK_TB), lambda i: (0, 2 * i + 1)),
            pl.BlockSpec((EMB_DIM, 4 * EMB_DIM), lambda i: (0, 0)),
        ],
        out_specs=pl.BlockSpec((_PACK_TB, 2 * EMB_DIM), lambda i: (i, 0)),
        out_shape=jax.ShapeDtypeStruct((_PACKED_ROWS, 2 * EMB_DIM),
                                       jnp.float32),
    )(tableT, tableT, eye)


_IDX_CHUNK = 128
_N_CHUNKS = _B_PER_W // _IDX_CHUNK


def _gather_pairs_sc(packed, pair_idx2d):
    mesh = plsc.VectorSubcoreMesh(core_axis_name="c", subcore_axis_name="s")

    @functools.partial(
        pl.kernel,
        mesh=mesh,
        out_type=jax.ShapeDtypeStruct((BATCH, 2 * EMB_DIM), jnp.float32),
        scratch_types=[
            pltpu.VMEM((_N_CHUNKS, _IDX_CHUNK), jnp.int32),
            pltpu.VMEM((_B_PER_W, 2 * EMB_DIM), jnp.float32),
            pltpu.SemaphoreType.DMA,
        ],
    )
    def gather_kernel(table_hbm, idx_hbm, out_hbm, idx_v, rows_v, sem):
        wid = lax.axis_index("s") * _SC_INFO.num_cores + lax.axis_index("c")
        base = wid * _B_PER_W
        pltpu.sync_copy(idx_hbm.at[pl.ds(wid * _N_CHUNKS, _N_CHUNKS)], idx_v)
        copies = []
        for c in range(_N_CHUNKS):
            copies.append(pltpu.async_copy(
                table_hbm.at[idx_v.at[c]],
                rows_v.at[pl.ds(c * _IDX_CHUNK, _IDX_CHUNK)],
                sem,
            ))
        for c in copies:
            c.wait()
        pltpu.sync_copy(rows_v, out_hbm.at[pl.ds(base, _B_PER_W)])

    return gather_kernel(packed, pair_idx2d)


def _mlp_body(pairs_ref, h_ref, featT_ref, w1a_ref, w1b_ref, b1_ref, w2_ref,
              b2_ref, w3_ref, b3_ref, outT_ref):
    cdim0 = (((0,), (0,)), ((), ()))
    tile = pairs_ref.shape[0] * 8
    pairs = pairs_ref[...].reshape(tile, 2 * EMB_DIM)
    hsel = h_ref[...]
    emb = (pairs[:, :EMB_DIM] * (1.0 - hsel)
           + pairs[:, EMB_DIM:] * hsel)
    h = jnp.dot(emb, w1a_ref[...], preferred_element_type=jnp.float32)
    h += lax.dot_general(featT_ref[...], w1b_ref[...], cdim0,
                         preferred_element_type=jnp.float32)
    h = jnp.maximum(h + b1_ref[...], 0.0)
    h = jnp.dot(h, w2_ref[...], preferred_element_type=jnp.float32)
    h = jnp.maximum(h + b2_ref[...], 0.0)
    o = jnp.dot(h, w3_ref[...], preferred_element_type=jnp.float32) + b3_ref[...]
    n = jnp.sqrt(jnp.sum(o * o, axis=-1, keepdims=True))
    outT_ref[...] = (o / jnp.maximum(n, 1e-12)).T


def _mlp_tc(pairs3, halff, featsT, W1, b1, W2, b2, W3, b3, tile=8192):
    grid = (BATCH // tile,)
    W1a = W1[:EMB_DIM]
    W1b = W1[EMB_DIM:]
    full = lambda r, c: pl.BlockSpec((r, c), lambda i: (0, 0))
    return pl.pallas_call(
        _mlp_body,
        grid=grid,
        in_specs=[
            pl.BlockSpec((tile // 8, 8, 2 * EMB_DIM), lambda i: (i, 0, 0)),
            pl.BlockSpec((tile, 1), lambda i: (i, 0)),
            pl.BlockSpec((FEAT_DIM, tile), lambda i: (0, i)),
            full(EMB_DIM, HID1),
            full(FEAT_DIM, HID1),
            full(1, HID1),
            full(HID1, HID2),
            full(1, HID2),
            full(HID2, OUT_DIM),
            full(1, OUT_DIM),
        ],
        out_specs=pl.BlockSpec((OUT_DIM, tile), lambda i: (0, i)),
        out_shape=jax.ShapeDtypeStruct((OUT_DIM, BATCH), jnp.float32),
    )(pairs3, halff, featsT, W1a, W1b, b1.reshape(1, HID1), W2,
      b2.reshape(1, HID2), W3, b3.reshape(1, OUT_DIM))


@jax.jit
def kernel(user_idx, user_feats, emb_table, W1, b1, W2, b2, W3, b3):
    idx = user_idx.astype(jnp.int32)
    eyes = jnp.concatenate(
        [jnp.eye(EMB_DIM, 2 * EMB_DIM, dtype=jnp.float32),
         jnp.eye(EMB_DIM, 2 * EMB_DIM, k=EMB_DIM, dtype=jnp.float32)], axis=1)
    packed = _pack_table_tc(emb_table.T, eyes)
    pack_row = (idx // (2 * _PACK_TB)) * _PACK_TB + (idx % _PACK_TB)
    halff = (((idx // _PACK_TB) % 2).astype(jnp.float32)).reshape(BATCH, 1)
    pairs = _gather_pairs_sc(
        packed,
        pack_row.reshape(_NUM_WORKERS * _N_CHUNKS, _IDX_CHUNK))
    pairs3 = pairs.reshape(BATCH // 8, 8, 2 * EMB_DIM)
    outT = _mlp_tc(pairs3, halff, user_feats.T, W1, b1, W2, b2, W3, b3)
    return outT.T


# speedup vs baseline: 1.0080x; 1.0080x over previous
import functools

import jax
import jax.numpy as jnp
from jax import lax
from jax.experimental import pallas as pl
from jax.experimental.pallas import tpu as pltpu
from jax.experimental.pallas import tpu_sc as plsc

NUM_USERS = 1000000
BATCH = 16384
EMB_DIM = 64
FEAT_DIM = 32
HID1 = 256
HID2 = 128
OUT_DIM = 64

_SC_INFO = plsc.get_sparse_core_info()
_NUM_WORKERS = _SC_INFO.num_cores * _SC_INFO.num_subcores
_B_PER_W = BATCH // _NUM_WORKERS

_PACK_TB = 16384


_N_PACK_BLOCKS = -(-NUM_USERS // (2 * _PACK_TB))
_PACKED_ROWS = _N_PACK_BLOCKS * _PACK_TB


def _pack_body(lo_ref, hi_ref, eye_ref, out_ref):
    cdim0 = (((0,), (0,)), ((), ()))
    eye = eye_ref[...]
    i = pl.program_id(0)
    valid = NUM_USERS - (2 * i + 1) * _PACK_TB
    cols = lax.broadcasted_iota(jnp.int32, (EMB_DIM, _PACK_TB), 1)
    hi = jnp.where(cols < valid, hi_ref[...], 0.0)
    out_ref[...] = (
        lax.dot_general(lo_ref[...], eye[:, :128], cdim0,
                        preferred_element_type=jnp.float32)
        + lax.dot_general(hi, eye[:, 128:], cdim0,
                          preferred_element_type=jnp.float32))


def _pack_table_tc(tableT, eye):
    return pl.pallas_call(
        _pack_body,
        grid=(_N_PACK_BLOCKS,),
        in_specs=[
            pl.BlockSpec((EMB_DIM, _PACK_TB), lambda i: (0, 2 * i)),
            pl.BlockSpec((EMB_DIM, _PACK_TB), lambda i: (0, 2 * i + 1)),
            pl.BlockSpec((EMB_DIM, 4 * EMB_DIM), lambda i: (0, 0)),
        ],
        out_specs=pl.BlockSpec((_PACK_TB, 2 * EMB_DIM), lambda i: (i, 0)),
        out_shape=jax.ShapeDtypeStruct((_PACKED_ROWS, 2 * EMB_DIM),
                                       jnp.float32),
    )(tableT, tableT, eye)


_IDX_CHUNK = 128
_N_CHUNKS = _B_PER_W // _IDX_CHUNK


def _gather_pairs_sc(packed, pair_idx2d):
    mesh = plsc.VectorSubcoreMesh(core_axis_name="c", subcore_axis_name="s")

    @functools.partial(
        pl.kernel,
        mesh=mesh,
        out_type=jax.ShapeDtypeStruct((BATCH, 2 * EMB_DIM), jnp.float32),
        scratch_types=[
            pltpu.VMEM((_N_CHUNKS, _IDX_CHUNK), jnp.int32),
            pltpu.VMEM((_B_PER_W, 2 * EMB_DIM), jnp.float32),
            pltpu.SemaphoreType.DMA,
        ],
    )
    def gather_kernel(table_hbm, idx_hbm, out_hbm, idx_v, rows_v, sem):
        wid = lax.axis_index("s") * _SC_INFO.num_cores + lax.axis_index("c")
        base = wid * _B_PER_W
        pltpu.sync_copy(idx_hbm.at[pl.ds(wid * _N_CHUNKS, _N_CHUNKS)], idx_v)
        copies = []
        for c in range(_N_CHUNKS):
            copies.append(pltpu.async_copy(
                table_hbm.at[idx_v.at[c]],
                rows_v.at[pl.ds(c * _IDX_CHUNK, _IDX_CHUNK)],
                sem,
            ))
        for c in copies:
            c.wait()
        pltpu.sync_copy(rows_v, out_hbm.at[pl.ds(base, _B_PER_W)])

    return gather_kernel(packed, pair_idx2d)


def _mlp_body(pairs_ref, h_ref, featT_ref, w1a_ref, w1b_ref, b1_ref, w2_ref,
              b2_ref, w3_ref, b3_ref, outT_ref):
    cdim0 = (((0,), (0,)), ((), ()))
    tile = pairs_ref.shape[0] * 8
    pairs = pairs_ref[...].reshape(tile, 2 * EMB_DIM)
    hsel = h_ref[...]
    emb = (pairs[:, :EMB_DIM] * (1.0 - hsel)
           + pairs[:, EMB_DIM:] * hsel)
    h = jnp.dot(emb, w1a_ref[...], preferred_element_type=jnp.float32)
    h += lax.dot_general(featT_ref[...], w1b_ref[...], cdim0,
                         preferred_element_type=jnp.float32)
    h = jnp.maximum(h + b1_ref[...], 0.0)
    h = jnp.dot(h, w2_ref[...], preferred_element_type=jnp.float32)
    h = jnp.maximum(h + b2_ref[...], 0.0)
    o = jnp.dot(h, w3_ref[...], preferred_element_type=jnp.float32) + b3_ref[...]
    n = jnp.sqrt(jnp.sum(o * o, axis=-1, keepdims=True))
    outT_ref[...] = (o / jnp.maximum(n, 1e-12)).T


def _mlp_tc(pairs3, halff, featsT, W1, b1, W2, b2, W3, b3, tile=4096):
    grid = (BATCH // tile,)
    W1a = W1[:EMB_DIM]
    W1b = W1[EMB_DIM:]
    full = lambda r, c: pl.BlockSpec((r, c), lambda i: (0, 0))
    return pl.pallas_call(
        _mlp_body,
        grid=grid,
        in_specs=[
            pl.BlockSpec((tile // 8, 8, 2 * EMB_DIM), lambda i: (i, 0, 0)),
            pl.BlockSpec((tile, 1), lambda i: (i, 0)),
            pl.BlockSpec((FEAT_DIM, tile), lambda i: (0, i)),
            full(EMB_DIM, HID1),
            full(FEAT_DIM, HID1),
            full(1, HID1),
            full(HID1, HID2),
            full(1, HID2),
            full(HID2, OUT_DIM),
            full(1, OUT_DIM),
        ],
        out_specs=pl.BlockSpec((OUT_DIM, tile), lambda i: (0, i)),
        out_shape=jax.ShapeDtypeStruct((OUT_DIM, BATCH), jnp.float32),
    )(pairs3, halff, featsT, W1a, W1b, b1.reshape(1, HID1), W2,
      b2.reshape(1, HID2), W3, b3.reshape(1, OUT_DIM))


@jax.jit
def kernel(user_idx, user_feats, emb_table, W1, b1, W2, b2, W3, b3):
    idx = user_idx.astype(jnp.int32)
    eyes = jnp.concatenate(
        [jnp.eye(EMB_DIM, 2 * EMB_DIM, dtype=jnp.float32),
         jnp.eye(EMB_DIM, 2 * EMB_DIM, k=EMB_DIM, dtype=jnp.float32)], axis=1)
    packed = _pack_table_tc(emb_table.T, eyes)
    pack_row = (idx // (2 * _PACK_TB)) * _PACK_TB + (idx % _PACK_TB)
    halff = (((idx // _PACK_TB) % 2).astype(jnp.float32)).reshape(BATCH, 1)
    pairs = _gather_pairs_sc(
        packed,
        pack_row.reshape(_NUM_WORKERS * _N_CHUNKS, _IDX_CHUNK))
    pairs3 = pairs.reshape(BATCH // 8, 8, 2 * EMB_DIM)
    outT = _mlp_tc(pairs3, halff, user_feats.T, W1, b1, W2, b2, W3, b3)
    return outT.T
